# Initial kernel scaffold; baseline (speedup 1.0000x reference)
import functools
import jax
import jax.numpy as jnp
from jax import lax
from jax.experimental import pallas as pl
from jax.experimental.pallas import tpu as pltpu

N = 10000
E = 160000
C = 128
H = 8
Dh = C // H
CUT = 5.0
EPS = 1e-08

BN = 1000
BE = 3200


def _silu(x):
    return x * jax.nn.sigmoid(x)


def _k1_body(nsr, nvr, lng, lnb, wnode, bnode, wcross,
             ns_out, nv_out, nva_out, nvb_out, nvc_out):
    x = nsr[...]
    mu = jnp.mean(x, axis=-1, keepdims=True)
    var = jnp.mean((x - mu) ** 2, axis=-1, keepdims=True)
    so = (x - mu) * lax.rsqrt(var + 1e-05) * lng[...] + lnb[...]
    ns_out[...] = lax.dot_general(so, wnode[...], (((1,), (1,)), ((), ())),
                                  preferred_element_type=jnp.float32) + bnode[...]

    vec = nvr[...]
    dist = jnp.sqrt(jnp.sum(vec * vec, axis=1, keepdims=True) + EPS)
    dist = jnp.maximum(dist, EPS)
    radial = vec / dist
    mn = jnp.min(dist, axis=-1, keepdims=True)
    mx = jnp.max(dist, axis=-1, keepdims=True)
    dn = (dist - mn) / jnp.maximum(mx - mn, EPS)
    nv = dn * radial
    nv_out[...] = nv
    nva_out[...] = jnp.concatenate([nv[:, 0, :64], nv[:, 1, :64], nv[:, 2, :64]],
                                   axis=-1)
    nvb_out[...] = jnp.concatenate([nv[:, 0, 64:], nv[:, 1, 64:], nv[:, 2, 64:]],
                                   axis=-1)
    b = nv.shape[0]
    nvf = nv.reshape(3 * b, C)
    nvc = lax.dot_general(nvf, wcross[...], (((1,), (1,)), ((), ())),
                          preferred_element_type=jnp.float32)
    nvc_out[...] = nvc.reshape(b, 3 * C)


def _k1(node_scalar, node_vector, ln_g, ln_b, W_node, b_node, W_cross):
    g = N // BN
    row = lambda i: (i, 0)
    row3 = lambda i: (i, 0, 0)
    full2 = lambda i: (0, 0)
    return pl.pallas_call(
        _k1_body,
        grid=(g,),
        in_specs=[
            pl.BlockSpec((BN, C), row),
            pl.BlockSpec((BN, 3, C), row3),
            pl.BlockSpec((1, C), full2),
            pl.BlockSpec((1, C), full2),
            pl.BlockSpec((C, C), full2),
            pl.BlockSpec((1, C), full2),
            pl.BlockSpec((C, C), full2),
        ],
        out_specs=[
            pl.BlockSpec((BN, C), row),
            pl.BlockSpec((BN, 3, C), row3),
            pl.BlockSpec((BN, 192), row),
            pl.BlockSpec((BN, 192), row),
            pl.BlockSpec((BN, 3 * C), row),
        ],
        out_shape=[
            jax.ShapeDtypeStruct((N, C), jnp.float32),
            jax.ShapeDtypeStruct((N, 3, C), jnp.float32),
            jax.ShapeDtypeStruct((N, 192), jnp.float32),
            jax.ShapeDtypeStruct((N, 192), jnp.float32),
            jax.ShapeDtypeStruct((N, 3 * C), jnp.float32),
        ],
    )(node_scalar, node_vector, ln_g.reshape(1, C), ln_b.reshape(1, C),
      W_node, b_node.reshape(1, C), W_cross)


def _k2_body(efr, dr, wedge, bedge, wf, bf, ef_out, fl_out, cw_out):
    x = efr[...]
    ef_out[...] = _silu(
        lax.dot_general(x, wedge[...], (((1,), (1,)), ((), ())),
                        preferred_element_type=jnp.float32) + bedge[...])
    fl_out[...] = _silu(
        lax.dot_general(x, wf[...], (((1,), (1,)), ((), ())),
                        preferred_element_type=jnp.float32) + bf[...])
    d = dr[...]
    cw_out[...] = 0.5 * (jnp.cos(d * (jnp.pi / CUT)) + 1.0) * (d < CUT).astype(jnp.float32)


def _run_k2(edge_feats, dist, W_edge, b_edge, W_f, b_f):
    g = E // BE
    row = lambda i: (i, 0)
    full2 = lambda i: (0, 0)
    return pl.pallas_call(
        _k2_body,
        grid=(g,),
        in_specs=[
            pl.BlockSpec((BE, C), row),
            pl.BlockSpec((BE // 128, 128), row),
            pl.BlockSpec((C, C), full2),
            pl.BlockSpec((1, C), full2),
            pl.BlockSpec((C, C), full2),
            pl.BlockSpec((1, C), full2),
        ],
        out_specs=[
            pl.BlockSpec((BE, C), row),
            pl.BlockSpec((BE, C), row),
            pl.BlockSpec((BE // 128, 128), row),
        ],
        out_shape=[
            jax.ShapeDtypeStruct((E, C), jnp.float32),
            jax.ShapeDtypeStruct((E, C), jnp.float32),
            jax.ShapeDtypeStruct((E // 128, 128), jnp.float32),
        ],
    )(edge_feats, dist.reshape(E // 128, 128), W_edge, b_edge.reshape(1, C),
      W_f, b_f.reshape(1, C))


def _k4_body(mr, wp1, bp1, pa_out, pb_out):
    p1 = _silu(
        lax.dot_general(mr[...], wp1[...], (((1,), (1,)), ((), ())),
                        preferred_element_type=jnp.float32) + bp1[...])
    pa_out[...] = jnp.concatenate([p1[:, 0:64], p1[:, 128:192]], axis=-1)
    pb_out[...] = jnp.concatenate([p1[:, 64:128], p1[:, 192:256]], axis=-1)


def _k4(m, W_p1, b_p1):
    g = E // BE
    row = lambda i: (i, 0)
    full2 = lambda i: (0, 0)
    return pl.pallas_call(
        _k4_body,
        grid=(g,),
        in_specs=[
            pl.BlockSpec((BE, C), row),
            pl.BlockSpec((2 * C, C), full2),
            pl.BlockSpec((1, 2 * C), full2),
        ],
        out_specs=[
            pl.BlockSpec((BE, C), row),
            pl.BlockSpec((BE, C), row),
        ],
        out_shape=[
            jax.ShapeDtypeStruct((E, C), jnp.float32),
            jax.ShapeDtypeStruct((E, C), jnp.float32),
        ],
    )(m, W_p1, b_p1.reshape(1, 2 * C))


def _k6_body(nsr, nvr, spr, voar, vobr, wvec, wp2, bp2, s_out, v_out):
    nsc = nsr[...] + spr[0] + spr[1]
    nv = nvr[...]
    voa = voar[...]
    vob = vobr[...]
    cols = []
    for k in range(3):
        cols.append(jnp.concatenate(
            [voa[:, 64 * k:64 * k + 64], vob[:, 64 * k:64 * k + 64]], axis=-1))
    vo = jnp.stack(cols, axis=1)
    nve = nv + vo
    b = nve.shape[0]
    vl = lax.dot_general(nve.reshape(3 * b, C), wvec[...],
                         (((1,), (1,)), ((), ())),
                         preferred_element_type=jnp.float32).reshape(b, 3, 2 * C)
    nv1 = vl[:, :, :C]
    nv2 = vl[:, :, C:]
    vec_tri = jnp.sum(nv1 * nv2, axis=1)
    norm = jnp.sqrt(jnp.sum(nv2 * nv2, axis=1) + 1e-08)
    qua = norm * norm * norm
    ps = lax.dot_general(nsc, wp2[...], (((1,), (1,)), ((), ())),
                         preferred_element_type=jnp.float32) + bp2[...]
    q1 = ps[:, :C]
    q2 = ps[:, C:2 * C]
    q3 = ps[:, 2 * C:]
    s_out[...] = nsc + (qua + vec_tri) * q1 + q2
    v_out[...] = nve + nv1 * q3[:, None, :]


def _k6(node_scalar, nv, sp, voa, vob, W_vec, W_p2, b_p2):
    g = N // BN
    row = lambda i: (i, 0)
    row3 = lambda i: (i, 0, 0)
    prow = lambda i: (0, i, 0)
    full2 = lambda i: (0, 0)
    return pl.pallas_call(
        _k6_body,
        grid=(g,),
        in_specs=[
            pl.BlockSpec((BN, C), row),
            pl.BlockSpec((BN, 3, C), row3),
            pl.BlockSpec((2, BN, C), prow),
            pl.BlockSpec((BN, 192), row),
            pl.BlockSpec((BN, 192), row),
            pl.BlockSpec((2 * C, C), full2),
            pl.BlockSpec((3 * C, C), full2),
            pl.BlockSpec((1, 3 * C), full2),
        ],
        out_specs=[
            pl.BlockSpec((BN, C), row),
            pl.BlockSpec((BN, 3, C), row3),
        ],
        out_shape=[
            jax.ShapeDtypeStruct((N, C), jnp.float32),
            jax.ShapeDtypeStruct((N, 3, C), jnp.float32),
        ],
    )(node_scalar, nv, sp, voa, vob, W_vec, W_p2, b_p2.reshape(1, 3 * C))


def _stage1_xla(ii, jj, ns, nvc_flat, ef, fl, edge_feats, cutw, ev, alpha):
    nsi = ns[ii]
    nsj = ns[jj]
    a = _silu(nsi + nsj + ef).reshape(E, H, Dh) * alpha
    attn = a.sum(axis=-1) * cutw[:, None]
    m = (nsj * ef).reshape(E, H, Dh) * attn[:, :, None]
    m = m.reshape(E, C)
    ai = nvc_flat[ii].reshape(E, 3, C)
    aj = nvc_flat[jj].reshape(E, 3, C)
    vv = jnp.sum(ev * ev, axis=-1)[:, None]
    ab = jnp.sum(ai * aj, axis=1)
    av = jnp.sum(ai * ev[:, :, None], axis=1)
    bv = jnp.sum(aj * ev[:, :, None], axis=1)
    sum_phi = ab * vv - av * bv
    eout = edge_feats + fl * sum_phi
    sp0 = jax.ops.segment_sum(m, ii, num_segments=N)
    sp = jnp.stack([sp0, jnp.zeros_like(sp0)])
    return m, eout, sp


def _stage2_xla(ii, jj, nva, nvb, pa, pb, ev):
    evr = jnp.repeat(ev, 64, axis=-1)
    voa = jax.ops.segment_sum(
        nva[jj] * jnp.tile(pa[:, :64], (1, 3)) + jnp.tile(pa[:, 64:], (1, 3)) * evr,
        ii, num_segments=N)
    vob = jax.ops.segment_sum(
        nvb[jj] * jnp.tile(pb[:, :64], (1, 3)) + jnp.tile(pb[:, 64:], (1, 3)) * evr,
        ii, num_segments=N)
    return voa, vob


def kernel(node_scalar, node_vector, edge_index, dist, edge_feats, edge_vector,
           ln_g, ln_b, alpha, W_vec, W_cross, W_node, b_node, W_edge, b_edge,
           W_p1, b_p1, W_p2, b_p2, W_f, b_f):
    ii = edge_index[0]
    jj = edge_index[1]
    ns, nv, nva, nvb, nvc_flat = _k1(node_scalar, node_vector, ln_g, ln_b,
                                     W_node, b_node, W_cross)
    ef, fl, cutw = _run_k2(edge_feats, dist, W_edge, b_edge, W_f, b_f)
    cutw = cutw.reshape(E)
    m, eout, sp = _stage1_xla(ii, jj, ns, nvc_flat, ef, fl, edge_feats, cutw,
                              edge_vector, alpha.reshape(1, H, Dh))
    pa, pb = _k4(m, W_p1, b_p1)
    voa, vob = _stage2_xla(ii, jj, nva, nvb, pa, pb, edge_vector)
    s_out, v_out = _k6(node_scalar, nv, sp, voa, vob, W_vec, W_p2, b_p2)
    return (s_out, v_out, eout)


# speedup vs baseline: 3.6753x; 3.6753x over previous
import functools
import jax
import jax.numpy as jnp
from jax import lax
from jax.experimental import pallas as pl
from jax.experimental.pallas import tpu as pltpu

N = 10000
E = 160000
C = 128
H = 8
Dh = C // H
CUT = 5.0
EPS = 1e-08

BN = 1000
BE = 3200


def _silu(x):
    return x * jax.nn.sigmoid(x)


def _k1_body(nsr, nvr, lng, lnb, wnode, bnode, wcross,
             ns_out, nv_out, nva_out, nvb_out, nvc_out):
    x = nsr[...]
    mu = jnp.mean(x, axis=-1, keepdims=True)
    var = jnp.mean((x - mu) ** 2, axis=-1, keepdims=True)
    so = (x - mu) * lax.rsqrt(var + 1e-05) * lng[...] + lnb[...]
    ns_out[...] = lax.dot_general(so, wnode[...], (((1,), (1,)), ((), ())),
                                  preferred_element_type=jnp.float32) + bnode[...]

    vec = nvr[...]
    dist = jnp.sqrt(jnp.sum(vec * vec, axis=1, keepdims=True) + EPS)
    dist = jnp.maximum(dist, EPS)
    radial = vec / dist
    mn = jnp.min(dist, axis=-1, keepdims=True)
    mx = jnp.max(dist, axis=-1, keepdims=True)
    dn = (dist - mn) / jnp.maximum(mx - mn, EPS)
    nv = dn * radial
    nv_out[...] = nv
    nva_out[...] = jnp.concatenate([nv[:, 0, :64], nv[:, 1, :64], nv[:, 2, :64]],
                                   axis=-1)
    nvb_out[...] = jnp.concatenate([nv[:, 0, 64:], nv[:, 1, 64:], nv[:, 2, 64:]],
                                   axis=-1)
    b = nv.shape[0]
    nvf = nv.reshape(3 * b, C)
    nvc = lax.dot_general(nvf, wcross[...], (((1,), (1,)), ((), ())),
                          preferred_element_type=jnp.float32)
    nvc_out[...] = nvc.reshape(b, 3 * C)


def _k1(node_scalar, node_vector, ln_g, ln_b, W_node, b_node, W_cross):
    g = N // BN
    row = lambda i: (i, 0)
    row3 = lambda i: (i, 0, 0)
    full2 = lambda i: (0, 0)
    return pl.pallas_call(
        _k1_body,
        grid=(g,),
        in_specs=[
            pl.BlockSpec((BN, C), row),
            pl.BlockSpec((BN, 3, C), row3),
            pl.BlockSpec((1, C), full2),
            pl.BlockSpec((1, C), full2),
            pl.BlockSpec((C, C), full2),
            pl.BlockSpec((1, C), full2),
            pl.BlockSpec((C, C), full2),
        ],
        out_specs=[
            pl.BlockSpec((BN, C), row),
            pl.BlockSpec((BN, 3, C), row3),
            pl.BlockSpec((BN, 192), row),
            pl.BlockSpec((BN, 192), row),
            pl.BlockSpec((BN, 3 * C), row),
        ],
        out_shape=[
            jax.ShapeDtypeStruct((N, C), jnp.float32),
            jax.ShapeDtypeStruct((N, 3, C), jnp.float32),
            jax.ShapeDtypeStruct((N, 192), jnp.float32),
            jax.ShapeDtypeStruct((N, 192), jnp.float32),
            jax.ShapeDtypeStruct((N, 3 * C), jnp.float32),
        ],
    )(node_scalar, node_vector, ln_g.reshape(1, C), ln_b.reshape(1, C),
      W_node, b_node.reshape(1, C), W_cross)


def _k2_body(efr, wedge, bedge, wf, bf, ef_out, fl_out):
    x = efr[...]
    ef_out[...] = _silu(
        lax.dot_general(x, wedge[...], (((1,), (1,)), ((), ())),
                        preferred_element_type=jnp.float32) + bedge[...])
    fl_out[...] = _silu(
        lax.dot_general(x, wf[...], (((1,), (1,)), ((), ())),
                        preferred_element_type=jnp.float32) + bf[...])


def _cutw_body(dr, cw_out):
    d = dr[...]
    cw_out[...] = 0.5 * (jnp.cos(d * (jnp.pi / CUT)) + 1.0) * (d < CUT).astype(jnp.float32)


def _cutw(dist):
    return pl.pallas_call(
        _cutw_body,
        out_shape=jax.ShapeDtypeStruct((E // 128, 128), jnp.float32),
    )(dist.reshape(E // 128, 128))


def _run_k2(edge_feats, dist, W_edge, b_edge, W_f, b_f):
    g = E // BE
    row = lambda i: (i, 0)
    full2 = lambda i: (0, 0)
    return pl.pallas_call(
        _k2_body,
        grid=(g,),
        in_specs=[
            pl.BlockSpec((BE, C), row),
            pl.BlockSpec((C, C), full2),
            pl.BlockSpec((1, C), full2),
            pl.BlockSpec((C, C), full2),
            pl.BlockSpec((1, C), full2),
        ],
        out_specs=[
            pl.BlockSpec((BE, C), row),
            pl.BlockSpec((BE, C), row),
        ],
        out_shape=[
            jax.ShapeDtypeStruct((E, C), jnp.float32),
            jax.ShapeDtypeStruct((E, C), jnp.float32),
        ],
    )(edge_feats, W_edge, b_edge.reshape(1, C), W_f, b_f.reshape(1, C))


def _k4_body(mr, wp1, bp1, pa_out, pb_out):
    p1 = _silu(
        lax.dot_general(mr[...], wp1[...], (((1,), (1,)), ((), ())),
                        preferred_element_type=jnp.float32) + bp1[...])
    pa_out[...] = jnp.concatenate([p1[:, 0:64], p1[:, 128:192]], axis=-1)
    pb_out[...] = jnp.concatenate([p1[:, 64:128], p1[:, 192:256]], axis=-1)


def _k4(m, W_p1, b_p1):
    g = E // BE
    row = lambda i: (i, 0)
    full2 = lambda i: (0, 0)
    return pl.pallas_call(
        _k4_body,
        grid=(g,),
        in_specs=[
            pl.BlockSpec((BE, C), row),
            pl.BlockSpec((2 * C, C), full2),
            pl.BlockSpec((1, 2 * C), full2),
        ],
        out_specs=[
            pl.BlockSpec((BE, C), row),
            pl.BlockSpec((BE, C), row),
        ],
        out_shape=[
            jax.ShapeDtypeStruct((E, C), jnp.float32),
            jax.ShapeDtypeStruct((E, C), jnp.float32),
        ],
    )(m, W_p1, b_p1.reshape(1, 2 * C))


def _k6_body(nsr, nvr, spr, voar, vobr, wvec, wp2, bp2, s_out, v_out):
    nsc = nsr[...] + spr[0] + spr[1]
    nv = nvr[...]
    voa = voar[...]
    vob = vobr[...]
    cols = []
    for k in range(3):
        cols.append(jnp.concatenate(
            [voa[:, 64 * k:64 * k + 64], vob[:, 64 * k:64 * k + 64]], axis=-1))
    vo = jnp.stack(cols, axis=1)
    nve = nv + vo
    b = nve.shape[0]
    vl = lax.dot_general(nve.reshape(3 * b, C), wvec[...],
                         (((1,), (1,)), ((), ())),
                         preferred_element_type=jnp.float32).reshape(b, 3, 2 * C)
    nv1 = vl[:, :, :C]
    nv2 = vl[:, :, C:]
    vec_tri = jnp.sum(nv1 * nv2, axis=1)
    norm = jnp.sqrt(jnp.sum(nv2 * nv2, axis=1) + 1e-08)
    qua = norm * norm * norm
    ps = lax.dot_general(nsc, wp2[...], (((1,), (1,)), ((), ())),
                         preferred_element_type=jnp.float32) + bp2[...]
    q1 = ps[:, :C]
    q2 = ps[:, C:2 * C]
    q3 = ps[:, 2 * C:]
    s_out[...] = nsc + (qua + vec_tri) * q1 + q2
    v_out[...] = nve + nv1 * q3[:, None, :]


def _k6(node_scalar, nv, sp, voa, vob, W_vec, W_p2, b_p2):
    g = N // BN
    row = lambda i: (i, 0)
    row3 = lambda i: (i, 0, 0)
    prow = lambda i: (0, i, 0)
    full2 = lambda i: (0, 0)
    return pl.pallas_call(
        _k6_body,
        grid=(g,),
        in_specs=[
            pl.BlockSpec((BN, C), row),
            pl.BlockSpec((BN, 3, C), row3),
            pl.BlockSpec((2, BN, C), prow),
            pl.BlockSpec((BN, 192), row),
            pl.BlockSpec((BN, 192), row),
            pl.BlockSpec((2 * C, C), full2),
            pl.BlockSpec((3 * C, C), full2),
            pl.BlockSpec((1, 3 * C), full2),
        ],
        out_specs=[
            pl.BlockSpec((BN, C), row),
            pl.BlockSpec((BN, 3, C), row3),
        ],
        out_shape=[
            jax.ShapeDtypeStruct((N, C), jnp.float32),
            jax.ShapeDtypeStruct((N, 3, C), jnp.float32),
        ],
    )(node_scalar, nv, sp, voa, vob, W_vec, W_p2, b_p2.reshape(1, 3 * C))


def _stage1_xla(ii, jj, ns, nvc_flat, ef, fl, edge_feats, cutw, ev, alpha):
    nsi = ns[ii]
    nsj = ns[jj]
    a = _silu(nsi + nsj + ef).reshape(E, H, Dh) * alpha
    attn = a.sum(axis=-1) * cutw[:, None]
    m = (nsj * ef).reshape(E, H, Dh) * attn[:, :, None]
    m = m.reshape(E, C)
    ai = nvc_flat[ii].reshape(E, 3, C)
    aj = nvc_flat[jj].reshape(E, 3, C)
    vv = jnp.sum(ev * ev, axis=-1)[:, None]
    ab = jnp.sum(ai * aj, axis=1)
    av = jnp.sum(ai * ev[:, :, None], axis=1)
    bv = jnp.sum(aj * ev[:, :, None], axis=1)
    sum_phi = ab * vv - av * bv
    eout = edge_feats + fl * sum_phi
    sp0 = jax.ops.segment_sum(m, ii, num_segments=N)
    sp = jnp.stack([sp0, jnp.zeros_like(sp0)])
    return m, eout, sp


def _stage2_xla(ii, jj, nva, nvb, pa, pb, ev):
    evr = jnp.repeat(ev, 64, axis=-1)
    voa = jax.ops.segment_sum(
        nva[jj] * jnp.tile(pa[:, :64], (1, 3)) + jnp.tile(pa[:, 64:], (1, 3)) * evr,
        ii, num_segments=N)
    vob = jax.ops.segment_sum(
        nvb[jj] * jnp.tile(pb[:, :64], (1, 3)) + jnp.tile(pb[:, 64:], (1, 3)) * evr,
        ii, num_segments=N)
    return voa, vob


def kernel(node_scalar, node_vector, edge_index, dist, edge_feats, edge_vector,
           ln_g, ln_b, alpha, W_vec, W_cross, W_node, b_node, W_edge, b_edge,
           W_p1, b_p1, W_p2, b_p2, W_f, b_f):
    ii = edge_index[0]
    jj = edge_index[1]
    ns, nv, nva, nvb, nvc_flat = _k1(node_scalar, node_vector, ln_g, ln_b,
                                     W_node, b_node, W_cross)
    ef, fl = _run_k2(edge_feats, dist, W_edge, b_edge, W_f, b_f)
    cutw = _cutw(dist).reshape(E)
    m, eout, sp = _stage1_xla(ii, jj, ns, nvc_flat, ef, fl, edge_feats, cutw,
                              edge_vector, alpha.reshape(1, H, Dh))
    pa, pb = _k4(m, W_p1, b_p1)
    voa, vob = _stage2_xla(ii, jj, nva, nvb, pa, pb, edge_vector)
    s_out, v_out = _k6(node_scalar, nv, sp, voa, vob, W_vec, W_p2, b_p2)
    return (s_out, v_out, eout)


# speedup vs baseline: 3.7335x; 1.0158x over previous
import functools
import jax
import jax.numpy as jnp
from jax import lax
from jax.experimental import pallas as pl
from jax.experimental.pallas import tpu as pltpu
from jax.experimental.pallas import tpu_sc as plsc

N = 10000
E = 160000
C = 128
H = 8
Dh = C // H
CUT = 5.0
EPS = 1e-08

BN = 1000
BE = 3200


def _silu(x):
    return x * jax.nn.sigmoid(x)


def _k1_body(nsr, nvr, lng, lnb, wnode, bnode, wcross,
             ns_out, nv_out, nva_out, nvb_out, nvc_out):
    x = nsr[...]
    mu = jnp.mean(x, axis=-1, keepdims=True)
    var = jnp.mean((x - mu) ** 2, axis=-1, keepdims=True)
    so = (x - mu) * lax.rsqrt(var + 1e-05) * lng[...] + lnb[...]
    ns_out[...] = lax.dot_general(so, wnode[...], (((1,), (1,)), ((), ())),
                                  preferred_element_type=jnp.float32) + bnode[...]

    vec = nvr[...]
    dist = jnp.sqrt(jnp.sum(vec * vec, axis=1, keepdims=True) + EPS)
    dist = jnp.maximum(dist, EPS)
    radial = vec / dist
    mn = jnp.min(dist, axis=-1, keepdims=True)
    mx = jnp.max(dist, axis=-1, keepdims=True)
    dn = (dist - mn) / jnp.maximum(mx - mn, EPS)
    nv = dn * radial
    nv_out[...] = nv
    nva_out[...] = jnp.concatenate([nv[:, 0, :64], nv[:, 1, :64], nv[:, 2, :64]],
                                   axis=-1)
    nvb_out[...] = jnp.concatenate([nv[:, 0, 64:], nv[:, 1, 64:], nv[:, 2, 64:]],
                                   axis=-1)
    b = nv.shape[0]
    nvf = nv.reshape(3 * b, C)
    nvc = lax.dot_general(nvf, wcross[...], (((1,), (1,)), ((), ())),
                          preferred_element_type=jnp.float32)
    nvc_out[...] = nvc.reshape(b, 3 * C)


def _k1(node_scalar, node_vector, ln_g, ln_b, W_node, b_node, W_cross):
    g = N // BN
    row = lambda i: (i, 0)
    row3 = lambda i: (i, 0, 0)
    full2 = lambda i: (0, 0)
    return pl.pallas_call(
        _k1_body,
        grid=(g,),
        in_specs=[
            pl.BlockSpec((BN, C), row),
            pl.BlockSpec((BN, 3, C), row3),
            pl.BlockSpec((1, C), full2),
            pl.BlockSpec((1, C), full2),
            pl.BlockSpec((C, C), full2),
            pl.BlockSpec((1, C), full2),
            pl.BlockSpec((C, C), full2),
        ],
        out_specs=[
            pl.BlockSpec((BN, C), row),
            pl.BlockSpec((BN, 3, C), row3),
            pl.BlockSpec((BN, 192), row),
            pl.BlockSpec((BN, 192), row),
            pl.BlockSpec((BN, 3 * C), row),
        ],
        out_shape=[
            jax.ShapeDtypeStruct((N, C), jnp.float32),
            jax.ShapeDtypeStruct((N, 3, C), jnp.float32),
            jax.ShapeDtypeStruct((N, 192), jnp.float32),
            jax.ShapeDtypeStruct((N, 192), jnp.float32),
            jax.ShapeDtypeStruct((N, 3 * C), jnp.float32),
        ],
    )(node_scalar, node_vector, ln_g.reshape(1, C), ln_b.reshape(1, C),
      W_node, b_node.reshape(1, C), W_cross)


def _k2_body(efr, wedge, bedge, wf, bf, ef_out, fl_out):
    x = efr[...]
    ef_out[...] = _silu(
        lax.dot_general(x, wedge[...], (((1,), (1,)), ((), ())),
                        preferred_element_type=jnp.float32) + bedge[...])
    fl_out[...] = _silu(
        lax.dot_general(x, wf[...], (((1,), (1,)), ((), ())),
                        preferred_element_type=jnp.float32) + bf[...])


def _cutw_body(dr, cw_out):
    d = dr[...]
    cw_out[...] = 0.5 * (jnp.cos(d * (jnp.pi / CUT)) + 1.0) * (d < CUT).astype(jnp.float32)


def _cutw(dist):
    return pl.pallas_call(
        _cutw_body,
        out_shape=jax.ShapeDtypeStruct((E // 128, 128), jnp.float32),
    )(dist.reshape(E // 128, 128))


def _run_k2(edge_feats, dist, W_edge, b_edge, W_f, b_f):
    g = E // BE
    row = lambda i: (i, 0)
    full2 = lambda i: (0, 0)
    return pl.pallas_call(
        _k2_body,
        grid=(g,),
        in_specs=[
            pl.BlockSpec((BE, C), row),
            pl.BlockSpec((C, C), full2),
            pl.BlockSpec((1, C), full2),
            pl.BlockSpec((C, C), full2),
            pl.BlockSpec((1, C), full2),
        ],
        out_specs=[
            pl.BlockSpec((BE, C), row),
            pl.BlockSpec((BE, C), row),
        ],
        out_shape=[
            jax.ShapeDtypeStruct((E, C), jnp.float32),
            jax.ShapeDtypeStruct((E, C), jnp.float32),
        ],
    )(edge_feats, W_edge, b_edge.reshape(1, C), W_f, b_f.reshape(1, C))


def _k4_body(mr, wp1, bp1, pa_out, pb_out):
    p1 = _silu(
        lax.dot_general(mr[...], wp1[...], (((1,), (1,)), ((), ())),
                        preferred_element_type=jnp.float32) + bp1[...])
    pa_out[...] = jnp.concatenate([p1[:, 0:64], p1[:, 128:192]], axis=-1)
    pb_out[...] = jnp.concatenate([p1[:, 64:128], p1[:, 192:256]], axis=-1)


def _k4(m, W_p1, b_p1):
    g = E // BE
    row = lambda i: (i, 0)
    full2 = lambda i: (0, 0)
    return pl.pallas_call(
        _k4_body,
        grid=(g,),
        in_specs=[
            pl.BlockSpec((BE, C), row),
            pl.BlockSpec((2 * C, C), full2),
            pl.BlockSpec((1, 2 * C), full2),
        ],
        out_specs=[
            pl.BlockSpec((BE, C), row),
            pl.BlockSpec((BE, C), row),
        ],
        out_shape=[
            jax.ShapeDtypeStruct((E, C), jnp.float32),
            jax.ShapeDtypeStruct((E, C), jnp.float32),
        ],
    )(m, W_p1, b_p1.reshape(1, 2 * C))


def _k6_body(nsr, nvr, spr, voar, vobr, wvec, wp2, bp2, s_out, v_out):
    nsc = nsr[...] + spr[0] + spr[1]
    nv = nvr[...]
    voa = voar[...]
    vob = vobr[...]
    cols = []
    for k in range(3):
        cols.append(jnp.concatenate(
            [voa[:, 64 * k:64 * k + 64], vob[:, 64 * k:64 * k + 64]], axis=-1))
    vo = jnp.stack(cols, axis=1)
    nve = nv + vo
    b = nve.shape[0]
    vl = lax.dot_general(nve.reshape(3 * b, C), wvec[...],
                         (((1,), (1,)), ((), ())),
                         preferred_element_type=jnp.float32).reshape(b, 3, 2 * C)
    nv1 = vl[:, :, :C]
    nv2 = vl[:, :, C:]
    vec_tri = jnp.sum(nv1 * nv2, axis=1)
    norm = jnp.sqrt(jnp.sum(nv2 * nv2, axis=1) + 1e-08)
    qua = norm * norm * norm
    ps = lax.dot_general(nsc, wp2[...], (((1,), (1,)), ((), ())),
                         preferred_element_type=jnp.float32) + bp2[...]
    q1 = ps[:, :C]
    q2 = ps[:, C:2 * C]
    q3 = ps[:, 2 * C:]
    s_out[...] = nsc + (qua + vec_tri) * q1 + q2
    v_out[...] = nve + nv1 * q3[:, None, :]


def _k6(node_scalar, nv, sp, voa, vob, W_vec, W_p2, b_p2):
    g = N // BN
    row = lambda i: (i, 0)
    row3 = lambda i: (i, 0, 0)
    prow = lambda i: (0, i, 0)
    full2 = lambda i: (0, 0)
    return pl.pallas_call(
        _k6_body,
        grid=(g,),
        in_specs=[
            pl.BlockSpec((BN, C), row),
            pl.BlockSpec((BN, 3, C), row3),
            pl.BlockSpec((2, BN, C), prow),
            pl.BlockSpec((BN, 192), row),
            pl.BlockSpec((BN, 192), row),
            pl.BlockSpec((2 * C, C), full2),
            pl.BlockSpec((3 * C, C), full2),
            pl.BlockSpec((1, 3 * C), full2),
        ],
        out_specs=[
            pl.BlockSpec((BN, C), row),
            pl.BlockSpec((BN, 3, C), row3),
        ],
        out_shape=[
            jax.ShapeDtypeStruct((N, C), jnp.float32),
            jax.ShapeDtypeStruct((N, 3, C), jnp.float32),
        ],
    )(node_scalar, nv, sp, voa, vob, W_vec, W_p2, b_p2.reshape(1, 3 * C))


_SC_CH = 128
_NCHUNK = E // _SC_CH
_ACC_R = 10240
_OWN = _ACC_R // 16
_ZR = 80


def _sc_scatter_body(m_hbm, ii_hbm, sp_hbm, idx_v, rows_v, stage_v, accum):
    c = lax.axis_index("c")
    s = lax.axis_index("s")
    w = s * 2 + c

    def zrow(r, carry):
        for q in range(8):
            stage_v[r, pl.ds(16 * q, 16)] = jnp.zeros((16,), jnp.float32)
        return carry
    lax.fori_loop(0, _ZR, zrow, 0)
    r0 = s * _OWN
    for z in range(_OWN // _ZR):
        pltpu.sync_copy(stage_v, accum.at[pl.ds(r0 + z * _ZR, _ZR)])
    plsc.subcore_barrier()

    def chunk(t, carry):
        cid = w + 32 * t

        @pl.when(cid < _NCHUNK)
        def _():
            base = cid * _SC_CH
            pltpu.sync_copy(ii_hbm.at[pl.ds(base, _SC_CH)], idx_v)
            pltpu.sync_copy(m_hbm.at[pl.ds(base, _SC_CH)], rows_v)
            pltpu.sync_copy(rows_v, accum.at[idx_v], add=True)
        return carry
    lax.fori_loop(0, (_NCHUNK + 31) // 32, chunk, 0)
    plsc.subcore_barrier()

    for z in range(_OWN // _ZR):
        row = r0 + z * _ZR

        @pl.when(row < N)
        def _():
            pltpu.sync_copy(accum.at[pl.ds(row, _ZR)], stage_v)
            pltpu.sync_copy(stage_v, sp_hbm.at[c, pl.ds(row, _ZR)])


def _sc_scatter(m, ii):
    return pl.kernel(
        _sc_scatter_body,
        out_type=jax.ShapeDtypeStruct((2, N, C), jnp.float32),
        mesh=plsc.VectorSubcoreMesh(core_axis_name="c", subcore_axis_name="s"),
        scratch_types=[
            pltpu.VMEM((_SC_CH,), jnp.int32),
            pltpu.VMEM((_SC_CH, C), jnp.float32),
            pltpu.VMEM((_ZR, C), jnp.float32),
            pltpu.VMEM_SHARED((_ACC_R, C), jnp.float32),
        ],
    )(m, ii)


def _stage1_xla(ii, jj, ns, nvc_flat, ef, fl, edge_feats, cutw, ev, alpha):
    nsi = ns[ii]
    nsj = ns[jj]
    a = _silu(nsi + nsj + ef).reshape(E, H, Dh) * alpha
    attn = a.sum(axis=-1) * cutw[:, None]
    m = (nsj * ef).reshape(E, H, Dh) * attn[:, :, None]
    m = m.reshape(E, C)
    ai = nvc_flat[ii].reshape(E, 3, C)
    aj = nvc_flat[jj].reshape(E, 3, C)
    vv = jnp.sum(ev * ev, axis=-1)[:, None]
    ab = jnp.sum(ai * aj, axis=1)
    av = jnp.sum(ai * ev[:, :, None], axis=1)
    bv = jnp.sum(aj * ev[:, :, None], axis=1)
    sum_phi = ab * vv - av * bv
    eout = edge_feats + fl * sum_phi
    return m, eout


def _stage2_xla(ii, jj, nva, nvb, pa, pb, ev):
    evr = jnp.repeat(ev, 64, axis=-1)
    voa = jax.ops.segment_sum(
        nva[jj] * jnp.tile(pa[:, :64], (1, 3)) + jnp.tile(pa[:, 64:], (1, 3)) * evr,
        ii, num_segments=N)
    vob = jax.ops.segment_sum(
        nvb[jj] * jnp.tile(pb[:, :64], (1, 3)) + jnp.tile(pb[:, 64:], (1, 3)) * evr,
        ii, num_segments=N)
    return voa, vob


def kernel(node_scalar, node_vector, edge_index, dist, edge_feats, edge_vector,
           ln_g, ln_b, alpha, W_vec, W_cross, W_node, b_node, W_edge, b_edge,
           W_p1, b_p1, W_p2, b_p2, W_f, b_f):
    ii = edge_index[0]
    jj = edge_index[1]
    ns, nv, nva, nvb, nvc_flat = _k1(node_scalar, node_vector, ln_g, ln_b,
                                     W_node, b_node, W_cross)
    ef, fl = _run_k2(edge_feats, dist, W_edge, b_edge, W_f, b_f)
    cutw = _cutw(dist).reshape(E)
    m, eout = _stage1_xla(ii, jj, ns, nvc_flat, ef, fl, edge_feats, cutw,
                          edge_vector, alpha.reshape(1, H, Dh))
    sp = _sc_scatter(m, ii)
    pa, pb = _k4(m, W_p1, b_p1)
    voa, vob = _stage2_xla(ii, jj, nva, nvb, pa, pb, edge_vector)
    s_out, v_out = _k6(node_scalar, nv, sp, voa, vob, W_vec, W_p2, b_p2)
    return (s_out, v_out, eout)
